# baseline (speedup 1.0000x reference)
import jax
import jax.numpy as jnp
from jax.experimental import pallas as pl
from jax.experimental.pallas import tpu as pltpu

_BB = 256
_BN = 8192


def _body(feat_ref, tfeat_ref, mem_ref, params_ref, out_t_ref, out_f_ref, sim_ref):
    t = params_ref[0, 0]
    inv_t = 1.0 / t
    f = feat_ref[...]
    tf = tfeat_ref[...]
    m = mem_ref[...]
    dims = (((1,), (1,)), ((), ()))
    out_f_ref[...] = jax.lax.dot_general(
        f, m, dims, preferred_element_type=jnp.float32) * inv_t
    out_t_ref[...] = jax.lax.dot_general(
        tf, m, dims, preferred_element_type=jnp.float32) * (inv_t * inv_t)

    @pl.when(pl.program_id(1) == 0)
    def _():
        sim_ref[...] = jnp.sum(f * tf, axis=-1, keepdims=True)


def kernel(image_features, transformed_image_features, indices, memory, params):
    del indices
    B, D = image_features.shape
    N = memory.shape[0]
    grid = (B // _BB, pl.cdiv(N, _BN))
    p2d = params.reshape(1, 2)
    out_t, out_f, sim = pl.pallas_call(
        _body,
        grid=grid,
        in_specs=[
            pl.BlockSpec((_BB, D), lambda i, j: (i, 0)),
            pl.BlockSpec((_BB, D), lambda i, j: (i, 0)),
            pl.BlockSpec((_BN, D), lambda i, j: (j, 0)),
            pl.BlockSpec((1, 2), lambda i, j: (0, 0)),
        ],
        out_specs=[
            pl.BlockSpec((_BB, _BN), lambda i, j: (i, j)),
            pl.BlockSpec((_BB, _BN), lambda i, j: (i, j)),
            pl.BlockSpec((_BB, 1), lambda i, j: (i, 0)),
        ],
        out_shape=[
            jax.ShapeDtypeStruct((B, N), jnp.float32),
            jax.ShapeDtypeStruct((B, N), jnp.float32),
            jax.ShapeDtypeStruct((B, 1), jnp.float32),
        ],
        compiler_params=pltpu.CompilerParams(
            dimension_semantics=("arbitrary", "arbitrary"),
        ),
    )(image_features, transformed_image_features, memory, p2d)
    return (out_t, out_f, sim)


# speedup vs baseline: 3.4964x; 3.4964x over previous
import jax
import jax.numpy as jnp
from jax.experimental import pallas as pl
from jax.experimental.pallas import tpu as pltpu

_BN = 2048


def _body(feat_ref, tfeat_ref, mem_ref, params_ref,
          out_t_ref, out_f_ref, sim_ref):
    t = params_ref[0, 0]
    inv_t = 1.0 / t
    f = feat_ref[...]
    tf = tfeat_ref[...]
    m = mem_ref[...]
    dims = (((1,), (1,)), ((), ()))
    out_f_ref[...] = jax.lax.dot_general(
        m, f, dims, preferred_element_type=jnp.float32) * inv_t
    out_t_ref[...] = jax.lax.dot_general(
        m, tf, dims, preferred_element_type=jnp.float32) * (inv_t * inv_t)

    @pl.when(pl.program_id(0) == 0)
    def _():
        sim_ref[...] = jnp.sum(f * tf, axis=-1, keepdims=True)


def kernel(image_features, transformed_image_features, indices, memory, params):
    del indices
    B, D = image_features.shape
    N = memory.shape[0]
    grid = (pl.cdiv(N, _BN),)
    p2d = params.reshape(1, 2)
    out_t, out_f, sim = pl.pallas_call(
        _body,
        grid=grid,
        in_specs=[
            pl.BlockSpec((B, D), lambda j: (0, 0)),
            pl.BlockSpec((B, D), lambda j: (0, 0)),
            pl.BlockSpec((_BN, D), lambda j: (j, 0)),
            pl.BlockSpec((1, 2), lambda j: (0, 0)),
        ],
        out_specs=[
            pl.BlockSpec((_BN, B), lambda j: (j, 0)),
            pl.BlockSpec((_BN, B), lambda j: (j, 0)),
            pl.BlockSpec((B, 1), lambda j: (0, 0)),
        ],
        out_shape=[
            jax.ShapeDtypeStruct((N, B), jnp.float32),
            jax.ShapeDtypeStruct((N, B), jnp.float32),
            jax.ShapeDtypeStruct((B, 1), jnp.float32),
        ],
        compiler_params=pltpu.CompilerParams(
            dimension_semantics=("parallel",),
        ),
    )(image_features, transformed_image_features, memory, p2d)
    return (out_t.T, out_f.T, sim)


# speedup vs baseline: 3.5121x; 1.0045x over previous
import jax
import jax.numpy as jnp
from jax.experimental import pallas as pl
from jax.experimental.pallas import tpu as pltpu

_BN = 3072


def _body(feat_ref, tfeat_ref, mem_ref, params_ref,
          out_t_ref, out_f_ref, sim_ref):
    t = params_ref[0, 0]
    inv_t = 1.0 / t
    f = feat_ref[...]
    tf = tfeat_ref[...]
    m = mem_ref[...]
    dims = (((1,), (1,)), ((), ()))
    out_f_ref[...] = jax.lax.dot_general(
        m, f, dims, preferred_element_type=jnp.float32) * inv_t
    out_t_ref[...] = jax.lax.dot_general(
        m, tf, dims, preferred_element_type=jnp.float32) * (inv_t * inv_t)

    @pl.when(pl.program_id(0) == 0)
    def _():
        sim_ref[...] = jnp.sum(f * tf, axis=-1, keepdims=True)


def kernel(image_features, transformed_image_features, indices, memory, params):
    del indices
    B, D = image_features.shape
    N = memory.shape[0]
    grid = (pl.cdiv(N, _BN),)
    p2d = params.reshape(1, 2)
    out_t, out_f, sim = pl.pallas_call(
        _body,
        grid=grid,
        in_specs=[
            pl.BlockSpec((B, D), lambda j: (0, 0)),
            pl.BlockSpec((B, D), lambda j: (0, 0)),
            pl.BlockSpec((_BN, D), lambda j: (j, 0)),
            pl.BlockSpec((1, 2), lambda j: (0, 0)),
        ],
        out_specs=[
            pl.BlockSpec((_BN, B), lambda j: (j, 0)),
            pl.BlockSpec((_BN, B), lambda j: (j, 0)),
            pl.BlockSpec((B, 1), lambda j: (0, 0)),
        ],
        out_shape=[
            jax.ShapeDtypeStruct((N, B), jnp.float32),
            jax.ShapeDtypeStruct((N, B), jnp.float32),
            jax.ShapeDtypeStruct((B, 1), jnp.float32),
        ],
        compiler_params=pltpu.CompilerParams(
            dimension_semantics=("parallel",),
        ),
    )(image_features, transformed_image_features, memory, p2d)
    return (out_t.T, out_f.T, sim)
